# initial kernel scaffold (unmeasured)
import jax
import jax.numpy as jnp
from jax import lax
from jax.experimental import pallas as pl
from jax.experimental.pallas import tpu as pltpu


def kernel(
    x,
):
    def body(*refs):
        pass

    out_shape = jax.ShapeDtypeStruct(..., jnp.float32)
    return pl.pallas_call(body, out_shape=out_shape)(...)


# baseline (device time: 120212 ns/iter reference)
import jax
import jax.numpy as jnp
from jax import lax
from jax.experimental import pallas as pl
from jax.experimental.pallas import tpu as pltpu


def kernel(x):
    xb = x.astype(jnp.bfloat16)
    m, n = xb.shape

    def body(x_ref, out_ref, recv_buf, send_sem, recv_sem):
        my_x = lax.axis_index("x")
        my_y = lax.axis_index("y")
        my_z = lax.axis_index("z")
        partner = (1 - my_x, my_y, my_z)

        barrier_sem = pltpu.get_barrier_semaphore()
        pl.semaphore_signal(
            barrier_sem, inc=1, device_id=partner,
            device_id_type=pl.DeviceIdType.MESH,
        )
        pl.semaphore_wait(barrier_sem, 1)

        rdma = pltpu.make_async_remote_copy(
            src_ref=x_ref,
            dst_ref=recv_buf,
            send_sem=send_sem,
            recv_sem=recv_sem,
            device_id=partner,
            device_id_type=pl.DeviceIdType.MESH,
        )
        rdma.start()
        rdma.wait()

        out_ref[...] = x_ref[...].astype(jnp.float32) + recv_buf[...].astype(
            jnp.float32
        )

    return pl.pallas_call(
        body,
        out_shape=jax.ShapeDtypeStruct((m, n), jnp.float32),
        in_specs=[pl.BlockSpec(memory_space=pltpu.VMEM)],
        out_specs=pl.BlockSpec(memory_space=pltpu.VMEM),
        scratch_shapes=[
            pltpu.VMEM((m, n), jnp.bfloat16),
            pltpu.SemaphoreType.DMA,
            pltpu.SemaphoreType.DMA,
        ],
        compiler_params=pltpu.CompilerParams(collective_id=0),
    )(xb)


# device time: 95365 ns/iter; 1.2605x vs baseline; 1.2605x over previous
import jax
import jax.numpy as jnp
from jax import lax
from jax.experimental import pallas as pl
from jax.experimental.pallas import tpu as pltpu

N_RING = 8


def kernel(x):
    xb = x.astype(jnp.bfloat16)
    m, n = xb.shape
    rows = m // N_RING

    def body(
        x_ref,
        out_ref,
        gbuf,
        p1recv,
        p1_send_sem,
        p1_recv_sem,
        cw_send_sems,
        cw_recv_sems,
        ccw_send_sems,
        ccw_recv_sems,
    ):
        my_x = lax.axis_index("x")
        my_y = lax.axis_index("y")
        my_z = lax.axis_index("z")
        partner = (1 - my_x, my_y, my_z)

        r = jnp.where(my_y == 0, my_z, 7 - my_z)

        def ring_coords(p):
            p = p % N_RING
            py = (p >= 4).astype(my_z.dtype)
            pz = jnp.where(p < 4, p, 7 - p)
            return (my_x, py, pz)

        nxt = ring_coords(r + 1)
        prv = ring_coords(r - 1)

        barrier_sem = pltpu.get_barrier_semaphore()
        for dev in (partner, nxt, prv):
            pl.semaphore_signal(
                barrier_sem, inc=1, device_id=dev,
                device_id_type=pl.DeviceIdType.MESH,
            )
        pl.semaphore_wait(barrier_sem, 3)

        my_rows = pl.ds(r * rows, rows)
        p1 = pltpu.make_async_remote_copy(
            src_ref=x_ref.at[my_rows],
            dst_ref=p1recv,
            send_sem=p1_send_sem,
            recv_sem=p1_recv_sem,
            device_id=partner,
            device_id_type=pl.DeviceIdType.MESH,
        )
        p1.start()
        p1.wait()
        s = x_ref[my_rows, :].astype(jnp.float32) + p1recv[...].astype(
            jnp.float32
        )
        out_ref[my_rows, :] = s
        gbuf[my_rows, :] = s.astype(jnp.bfloat16)

        def chunk(k):
            return pl.ds((k % N_RING) * rows, rows)

        for h in range(N_RING // 2):
            send_cw = pltpu.make_async_remote_copy(
                src_ref=gbuf.at[chunk(r - h)],
                dst_ref=gbuf.at[chunk(r - h)],
                send_sem=cw_send_sems.at[h],
                recv_sem=cw_recv_sems.at[h],
                device_id=nxt,
                device_id_type=pl.DeviceIdType.MESH,
            )
            send_cw.start()
            if h < N_RING // 2 - 1:
                send_ccw = pltpu.make_async_remote_copy(
                    src_ref=gbuf.at[chunk(r + h)],
                    dst_ref=gbuf.at[chunk(r + h)],
                    send_sem=ccw_send_sems.at[h],
                    recv_sem=ccw_recv_sems.at[h],
                    device_id=prv,
                    device_id_type=pl.DeviceIdType.MESH,
                )
                send_ccw.start()

            recv_cw = pltpu.make_async_remote_copy(
                src_ref=gbuf.at[chunk(r - 1 - h)],
                dst_ref=gbuf.at[chunk(r - 1 - h)],
                send_sem=cw_send_sems.at[h],
                recv_sem=cw_recv_sems.at[h],
                device_id=nxt,
                device_id_type=pl.DeviceIdType.MESH,
            )
            recv_cw.wait_recv()
            send_cw.wait_send()
            out_ref[chunk(r - 1 - h), :] = gbuf[chunk(r - 1 - h), :].astype(
                jnp.float32
            )
            if h < N_RING // 2 - 1:
                recv_ccw = pltpu.make_async_remote_copy(
                    src_ref=gbuf.at[chunk(r + 1 + h)],
                    dst_ref=gbuf.at[chunk(r + 1 + h)],
                    send_sem=ccw_send_sems.at[h],
                    recv_sem=ccw_recv_sems.at[h],
                    device_id=prv,
                    device_id_type=pl.DeviceIdType.MESH,
                )
                recv_ccw.wait_recv()
                send_ccw.wait_send()
                out_ref[chunk(r + 1 + h), :] = gbuf[
                    chunk(r + 1 + h), :
                ].astype(jnp.float32)

    return pl.pallas_call(
        body,
        out_shape=jax.ShapeDtypeStruct((m, n), jnp.float32),
        in_specs=[pl.BlockSpec(memory_space=pltpu.VMEM)],
        out_specs=pl.BlockSpec(memory_space=pltpu.VMEM),
        scratch_shapes=[
            pltpu.VMEM((m, n), jnp.bfloat16),
            pltpu.VMEM((rows, n), jnp.bfloat16),
            pltpu.SemaphoreType.DMA,
            pltpu.SemaphoreType.DMA,
            pltpu.SemaphoreType.DMA((N_RING // 2,)),
            pltpu.SemaphoreType.DMA((N_RING // 2,)),
            pltpu.SemaphoreType.DMA((N_RING // 2 - 1,)),
            pltpu.SemaphoreType.DMA((N_RING // 2 - 1,)),
        ],
        compiler_params=pltpu.CompilerParams(collective_id=0),
    )(xb)


# device time: 73505 ns/iter; 1.6354x vs baseline; 1.2974x over previous
import jax
import jax.numpy as jnp
from jax import lax
from jax.experimental import pallas as pl
from jax.experimental.pallas import tpu as pltpu

N_RING = 8


def kernel(x):
    m, n = x.shape
    rows = m // N_RING

    def body(
        x_hbm,
        out_ref,
        xchunk,
        mysend,
        p1recv,
        copy_sem,
        p1_send_sem,
        p1_recv_sem,
        cw_send_sems,
        cw_recv_sems,
        ccw_send_sems,
        ccw_recv_sems,
    ):
        my_x = lax.axis_index("x")
        my_y = lax.axis_index("y")
        my_z = lax.axis_index("z")
        partner = (1 - my_x, my_y, my_z)

        r = jnp.where(my_y == 0, my_z, 7 - my_z)

        def ring_coords(p):
            p = p % N_RING
            py = (p >= 4).astype(my_z.dtype)
            pz = jnp.where(p < 4, p, 7 - p)
            return (my_x, py, pz)

        nxt = ring_coords(r + 1)
        prv = ring_coords(r - 1)

        my_rows = pl.ds(r * rows, rows)
        cp = pltpu.make_async_copy(x_hbm.at[my_rows], xchunk, copy_sem)
        cp.start()

        barrier_sem = pltpu.get_barrier_semaphore()
        for dev in (partner, nxt, prv):
            pl.semaphore_signal(
                barrier_sem, inc=1, device_id=dev,
                device_id_type=pl.DeviceIdType.MESH,
            )
        pl.semaphore_wait(barrier_sem, 3)

        cp.wait()
        mysend[...] = xchunk[...].astype(jnp.bfloat16)

        p1 = pltpu.make_async_remote_copy(
            src_ref=mysend,
            dst_ref=p1recv,
            send_sem=p1_send_sem,
            recv_sem=p1_recv_sem,
            device_id=partner,
            device_id_type=pl.DeviceIdType.MESH,
        )
        p1.start()
        p1.wait()
        out_ref[my_rows, :] = (
            xchunk[...] + p1recv[...].astype(jnp.float32)
        ).astype(jnp.bfloat16)

        def chunk(k):
            return pl.ds((k % N_RING) * rows, rows)

        for h in range(N_RING // 2):
            send_cw = pltpu.make_async_remote_copy(
                src_ref=out_ref.at[chunk(r - h)],
                dst_ref=out_ref.at[chunk(r - h)],
                send_sem=cw_send_sems.at[h],
                recv_sem=cw_recv_sems.at[h],
                device_id=nxt,
                device_id_type=pl.DeviceIdType.MESH,
            )
            send_cw.start()
            if h < N_RING // 2 - 1:
                send_ccw = pltpu.make_async_remote_copy(
                    src_ref=out_ref.at[chunk(r + h)],
                    dst_ref=out_ref.at[chunk(r + h)],
                    send_sem=ccw_send_sems.at[h],
                    recv_sem=ccw_recv_sems.at[h],
                    device_id=prv,
                    device_id_type=pl.DeviceIdType.MESH,
                )
                send_ccw.start()

            recv_cw = pltpu.make_async_remote_copy(
                src_ref=out_ref.at[chunk(r - 1 - h)],
                dst_ref=out_ref.at[chunk(r - 1 - h)],
                send_sem=cw_send_sems.at[h],
                recv_sem=cw_recv_sems.at[h],
                device_id=nxt,
                device_id_type=pl.DeviceIdType.MESH,
            )
            recv_cw.wait_recv()
            send_cw.wait_send()
            if h < N_RING // 2 - 1:
                recv_ccw = pltpu.make_async_remote_copy(
                    src_ref=out_ref.at[chunk(r + 1 + h)],
                    dst_ref=out_ref.at[chunk(r + 1 + h)],
                    send_sem=ccw_send_sems.at[h],
                    recv_sem=ccw_recv_sems.at[h],
                    device_id=prv,
                    device_id_type=pl.DeviceIdType.MESH,
                )
                recv_ccw.wait_recv()
                send_ccw.wait_send()

    return pl.pallas_call(
        body,
        out_shape=jax.ShapeDtypeStruct((m, n), jnp.bfloat16),
        in_specs=[pl.BlockSpec(memory_space=pl.ANY)],
        out_specs=pl.BlockSpec(memory_space=pltpu.VMEM),
        scratch_shapes=[
            pltpu.VMEM((rows, n), jnp.float32),
            pltpu.VMEM((rows, n), jnp.bfloat16),
            pltpu.VMEM((rows, n), jnp.bfloat16),
            pltpu.SemaphoreType.DMA,
            pltpu.SemaphoreType.DMA,
            pltpu.SemaphoreType.DMA,
            pltpu.SemaphoreType.DMA((N_RING // 2,)),
            pltpu.SemaphoreType.DMA((N_RING // 2,)),
            pltpu.SemaphoreType.DMA((N_RING // 2 - 1,)),
            pltpu.SemaphoreType.DMA((N_RING // 2 - 1,)),
        ],
        compiler_params=pltpu.CompilerParams(collective_id=0),
    )(x)


# device time: 58049 ns/iter; 2.0709x vs baseline; 1.2663x over previous
import jax
import jax.numpy as jnp
from jax import lax
from jax.experimental import pallas as pl
from jax.experimental.pallas import tpu as pltpu

N_RING = 8


def kernel(x):
    m, n = x.shape
    rows = m // N_RING
    hrows = rows // 2

    def body(
        x_hbm,
        out_ref,
        xchunk,
        mysend,
        p1recv,
        copy_sem,
        p1_send_sems,
        p1_recv_sems,
        cw_a_send, cw_a_recv,
        cw_b_send, cw_b_recv,
        ccw_b_send, ccw_b_recv,
        ccw_a_send, ccw_a_recv,
    ):
        my_x = lax.axis_index("x")
        my_y = lax.axis_index("y")
        my_z = lax.axis_index("z")
        partner = (1 - my_x, my_y, my_z)

        r = jnp.where(my_y == 0, my_z, 7 - my_z)

        def ring_coords(p):
            p = p % N_RING
            py = (p >= 4).astype(my_z.dtype)
            pz = jnp.where(p < 4, p, 7 - p)
            return (my_x, py, pz)

        nxt = ring_coords(r + 1)
        prv = ring_coords(r - 1)

        def hs(k, hf):
            return pl.ds((k % N_RING) * rows + hf * hrows, hrows)

        my_rows = pl.ds(r * rows, rows)
        cp = pltpu.make_async_copy(x_hbm.at[my_rows], xchunk, copy_sem)
        cp.start()

        barrier_sem = pltpu.get_barrier_semaphore()
        for dev in (partner, nxt, prv):
            pl.semaphore_signal(
                barrier_sem, inc=1, device_id=dev,
                device_id_type=pl.DeviceIdType.MESH,
            )
        pl.semaphore_wait(barrier_sem, 3)

        cp.wait()
        mysend[...] = xchunk[...].astype(jnp.bfloat16)

        sends = []

        def rcopy(slc, send_sems, recv_sems, idx, dev):
            return pltpu.make_async_remote_copy(
                src_ref=out_ref.at[slc],
                dst_ref=out_ref.at[slc],
                send_sem=send_sems.at[idx],
                recv_sem=recv_sems.at[idx],
                device_id=dev,
                device_id_type=pl.DeviceIdType.MESH,
            )

        p1 = []
        for hf in (0, 1):
            d = pltpu.make_async_remote_copy(
                src_ref=mysend.at[pl.ds(hf * hrows, hrows)],
                dst_ref=p1recv.at[pl.ds(hf * hrows, hrows)],
                send_sem=p1_send_sems.at[hf],
                recv_sem=p1_recv_sems.at[hf],
                device_id=partner,
                device_id_type=pl.DeviceIdType.MESH,
            )
            d.start()
            p1.append(d)
            sends.append(d)

        for hf in (0, 1):
            p1[hf].wait_recv()
            src = pl.ds(hf * hrows, hrows)
            out_ref[hs(r, hf), :] = (
                xchunk[src, :] + p1recv[src, :].astype(jnp.float32)
            ).astype(jnp.bfloat16)
            if hf == 0:
                d = rcopy(hs(r, 0), cw_a_send, cw_a_recv, 0, nxt)
                d.start(); sends.append(d)
                d = rcopy(hs(r, 0), ccw_a_send, ccw_a_recv, 0, prv)
                d.start(); sends.append(d)
            else:
                d = rcopy(hs(r, 1), cw_b_send, cw_b_recv, 0, nxt)
                d.start(); sends.append(d)
                d = rcopy(hs(r, 1), ccw_b_send, ccw_b_recv, 0, prv)
                d.start(); sends.append(d)

        for j in range(4):
            rd = rcopy(hs(r - 1 - j, 0), cw_a_send, cw_a_recv, j, nxt)
            rd.wait_recv()
            if j + 1 < 4:
                d = rcopy(hs(r - 1 - j, 0), cw_a_send, cw_a_recv, j + 1, nxt)
                d.start(); sends.append(d)
            if j < 3:
                rd = rcopy(hs(r - 1 - j, 1), cw_b_send, cw_b_recv, j, nxt)
                rd.wait_recv()
                if j + 1 < 3:
                    d = rcopy(
                        hs(r - 1 - j, 1), cw_b_send, cw_b_recv, j + 1, nxt
                    )
                    d.start(); sends.append(d)
            rd = rcopy(hs(r + 1 + j, 1), ccw_b_send, ccw_b_recv, j, prv)
            rd.wait_recv()
            if j + 1 < 4:
                d = rcopy(hs(r + 1 + j, 1), ccw_b_send, ccw_b_recv, j + 1, prv)
                d.start(); sends.append(d)
            if j < 3:
                rd = rcopy(hs(r + 1 + j, 0), ccw_a_send, ccw_a_recv, j, prv)
                rd.wait_recv()
                if j + 1 < 3:
                    d = rcopy(
                        hs(r + 1 + j, 0), ccw_a_send, ccw_a_recv, j + 1, prv
                    )
                    d.start(); sends.append(d)

        for d in sends:
            d.wait_send()

    return pl.pallas_call(
        body,
        out_shape=jax.ShapeDtypeStruct((m, n), jnp.bfloat16),
        in_specs=[pl.BlockSpec(memory_space=pl.ANY)],
        out_specs=pl.BlockSpec(memory_space=pltpu.VMEM),
        scratch_shapes=[
            pltpu.VMEM((rows, n), jnp.float32),
            pltpu.VMEM((rows, n), jnp.bfloat16),
            pltpu.VMEM((rows, n), jnp.bfloat16),
            pltpu.SemaphoreType.DMA,
            pltpu.SemaphoreType.DMA((2,)),
            pltpu.SemaphoreType.DMA((2,)),
            pltpu.SemaphoreType.DMA((4,)), pltpu.SemaphoreType.DMA((4,)),
            pltpu.SemaphoreType.DMA((3,)), pltpu.SemaphoreType.DMA((3,)),
            pltpu.SemaphoreType.DMA((4,)), pltpu.SemaphoreType.DMA((4,)),
            pltpu.SemaphoreType.DMA((3,)), pltpu.SemaphoreType.DMA((3,)),
        ],
        compiler_params=pltpu.CompilerParams(collective_id=0),
    )(x)
